# baseline (speedup 1.0000x reference)
import functools

import jax
import jax.numpy as jnp
from jax import lax
from jax.experimental import pallas as pl
from jax.experimental.pallas import tpu as pltpu
from jax.experimental.pallas import tpu_sc as plsc

NC = 2
NS = 16
NW = NC * NS
L = 16


def _proj_body(x_ref, w_ref, al_ref, ar_ref, g_ref):
    w = w_ref[...]
    wl = w[:, 0:1] * al_ref[0] + w[:, 1:2] * al_ref[1]
    wr = w[:, 0:1] * ar_ref[0] + w[:, 1:2] * ar_ref[1]
    wcat = jnp.concatenate([w, wl, wr], axis=1)
    g_ref[...] = lax.dot_general(
        wcat, x_ref[...], (((0,), (1,)), ((), ())),
        preferred_element_type=jnp.float32)


def _edge_body(n, npad, e, e_per_w, g_hbm, ei_hbm, out_hbm,
               g_v, acc_v, src_v, dst_v, sem):
    cid = lax.axis_index("c")
    sid = lax.axis_index("s")
    wid = sid * NC + cid
    base = wid * e_per_w

    c1 = pltpu.async_copy(ei_hbm.at[pl.ds(base, e_per_w)], src_v, sem)
    c2 = pltpu.async_copy(ei_hbm.at[pl.ds(e + base, e_per_w)], dst_v, sem)
    c3 = pltpu.async_copy(g_hbm, g_v, sem)

    @plsc.parallel_loop(0, 3 * npad, step=L, unroll=8)
    def _zero(i):
        acc_v[pl.ds(i, L)] = jnp.zeros((L,), jnp.float32)

    c1.wait()
    c2.wait()
    c3.wait()

    @plsc.parallel_loop(0, e_per_w, step=L, unroll=8)
    def edge_group(i):
        s = src_v[pl.ds(i, L)]
        d = dst_v[pl.ds(i, L)]
        h0 = plsc.load_gather(g_v, [s])
        h1 = plsc.load_gather(g_v, [s + n])
        el = plsc.load_gather(g_v, [s + 2 * n])
        er = plsc.load_gather(g_v, [d + 3 * n])
        ee = el + er
        ee = jnp.where(ee >= 0.0, ee, ee * 0.2)
        w = jnp.exp(ee)
        plsc.addupdate_scatter(acc_v, [d], w)
        plsc.addupdate_scatter(acc_v, [d + npad], w * h0)
        plsc.addupdate_scatter(acc_v, [d + 2 * npad], w * h1)

    pltpu.sync_copy(acc_v, out_hbm.at[wid])


def _final_body(x_ref, nz_ref, b_ref, p_ref, o_ref):
    den = jnp.sum(p_ref[:, 0, :], axis=0, keepdims=True) + 1e-9
    n0 = jnp.sum(p_ref[:, 1, :], axis=0, keepdims=True)
    n1 = jnp.sum(p_ref[:, 2, :], axis=0, keepdims=True)
    std = jnp.maximum(n0 / den + b_ref[0], 0.0)
    mean = jnp.maximum(n1 / den + b_ref[1], 0.0)
    z = nz_ref[...] * std.T + mean.T
    gate = 1.0 / (1.0 + jnp.exp(-z))
    o_ref[...] = x_ref[...] * gate


def kernel(input, edge_index, degree, W, attn_l, attn_r, bias, noise_x):
    x = input
    n, d = x.shape
    e = edge_index.shape[1]
    ei_flat = edge_index.astype(jnp.int32).reshape(-1)

    bn = 2048
    grid = pl.cdiv(n, bn)
    npad = bn * grid
    e_per_w = e // NW

    g = pl.pallas_call(
        _proj_body,
        grid=(grid,),
        in_specs=[
            pl.BlockSpec((bn, d), lambda i: (i, 0)),
            pl.BlockSpec((d, 2), lambda i: (0, 0)),
            pl.BlockSpec(memory_space=pltpu.SMEM),
            pl.BlockSpec(memory_space=pltpu.SMEM),
        ],
        out_specs=pl.BlockSpec((4, bn), lambda i: (0, i)),
        out_shape=jax.ShapeDtypeStruct((4, n), jnp.float32),
    )(x, W, attn_l, attn_r)

    mesh = plsc.VectorSubcoreMesh(core_axis_name="c", subcore_axis_name="s")
    partials = pl.kernel(
        functools.partial(_edge_body, n, npad, e, e_per_w),
        out_type=jax.ShapeDtypeStruct((NW, 3 * npad), jnp.float32),
        mesh=mesh,
        scratch_types=[
            pltpu.VMEM((4 * n,), jnp.float32),
            pltpu.VMEM((3 * npad,), jnp.float32),
            pltpu.VMEM((e_per_w,), jnp.int32),
            pltpu.VMEM((e_per_w,), jnp.int32),
            pltpu.SemaphoreType.DMA,
        ],
        compiler_params=pltpu.CompilerParams(needs_layout_passes=False),
    )(g.reshape(-1), ei_flat)

    out = pl.pallas_call(
        _final_body,
        grid=(grid,),
        in_specs=[
            pl.BlockSpec((bn, d), lambda i: (i, 0)),
            pl.BlockSpec((bn, 1), lambda i: (i, 0)),
            pl.BlockSpec(memory_space=pltpu.SMEM),
            pl.BlockSpec((NW, 3, bn), lambda i: (0, 0, i)),
        ],
        out_specs=pl.BlockSpec((bn, d), lambda i: (i, 0)),
        out_shape=jax.ShapeDtypeStruct((n, d), jnp.float32),
    )(x, noise_x, bias, partials.reshape(NW, 3, npad))
    return out


# speedup vs baseline: 1.1160x; 1.1160x over previous
import functools

import jax
import jax.numpy as jnp
from jax import lax
from jax.experimental import pallas as pl
from jax.experimental.pallas import tpu as pltpu
from jax.experimental.pallas import tpu_sc as plsc

NC = 2
NS = 16
NW = NC * NS
L = 16


def _proj_body(x_ref, w_ref, al_ref, ar_ref, g_ref):
    w = w_ref[...]
    wl = w[:, 0:1] * al_ref[0] + w[:, 1:2] * al_ref[1]
    wr = w[:, 0:1] * ar_ref[0] + w[:, 1:2] * ar_ref[1]
    wcat = jnp.concatenate([w, wl, wr], axis=1)
    g_ref[...] = lax.dot_general(
        wcat, x_ref[...], (((0,), (1,)), ((), ())),
        preferred_element_type=jnp.float32)


def _edge_body(n, npad, e, e_per_w, g_hbm, ei_hbm, out_hbm,
               g_v, acc_v, src_v, dst_v, sem):
    cid = lax.axis_index("c")
    sid = lax.axis_index("s")
    wid = sid * NC + cid
    base = wid * e_per_w

    c1 = pltpu.async_copy(ei_hbm.at[pl.ds(base, e_per_w)], src_v, sem)
    c2 = pltpu.async_copy(ei_hbm.at[pl.ds(e + base, e_per_w)], dst_v, sem)
    c3 = pltpu.async_copy(g_hbm, g_v, sem)

    @plsc.parallel_loop(0, 3 * npad, step=L, unroll=8)
    def _zero(i):
        acc_v[pl.ds(i, L)] = jnp.zeros((L,), jnp.float32)

    c1.wait()
    c2.wait()
    c3.wait()

    @plsc.parallel_loop(0, e_per_w, step=L, unroll=8)
    def edge_group(i):
        s = src_v[pl.ds(i, L)]
        d = dst_v[pl.ds(i, L)]
        h0 = plsc.load_gather(g_v, [s])
        h1 = plsc.load_gather(g_v, [s + n])
        el = plsc.load_gather(g_v, [s + 2 * n])
        er = plsc.load_gather(g_v, [d + 3 * n])
        ee = el + er
        ee = jnp.where(ee >= 0.0, ee, ee * 0.2)
        w = jnp.exp(ee)
        plsc.addupdate_scatter(acc_v, [d], w)
        plsc.addupdate_scatter(acc_v, [d + npad], w * h0)
        plsc.addupdate_scatter(acc_v, [d + 2 * npad], w * h1)

    pltpu.sync_copy(acc_v, out_hbm.at[wid])


def _final_body(npad, bn, x_ref, nz_ref, b_ref, p_ref, o_ref):
    col = pl.program_id(0) * bn
    den = jnp.sum(p_ref[:, pl.ds(col, bn)], axis=0,
                  keepdims=True) + 1e-9
    n0 = jnp.sum(p_ref[:, pl.ds(col + npad, bn)], axis=0, keepdims=True)
    n1 = jnp.sum(p_ref[:, pl.ds(col + 2 * npad, bn)], axis=0, keepdims=True)
    std = jnp.maximum(n0 / den + b_ref[0], 0.0)
    mean = jnp.maximum(n1 / den + b_ref[1], 0.0)
    z = nz_ref[...] * std.T + mean.T
    gate = 1.0 / (1.0 + jnp.exp(-z))
    o_ref[...] = x_ref[...] * gate


def kernel(input, edge_index, degree, W, attn_l, attn_r, bias, noise_x):
    x = input
    n, d = x.shape
    e = edge_index.shape[1]
    ei_flat = edge_index.astype(jnp.int32).reshape(-1)

    bn = 2048
    grid = pl.cdiv(n, bn)
    npad = bn * grid
    e_per_w = e // NW

    g = pl.pallas_call(
        _proj_body,
        grid=(grid,),
        in_specs=[
            pl.BlockSpec((bn, d), lambda i: (i, 0)),
            pl.BlockSpec((d, 2), lambda i: (0, 0)),
            pl.BlockSpec(memory_space=pltpu.SMEM),
            pl.BlockSpec(memory_space=pltpu.SMEM),
        ],
        out_specs=pl.BlockSpec((4, bn), lambda i: (0, i)),
        out_shape=jax.ShapeDtypeStruct((4, n), jnp.float32),
    )(x, W, attn_l, attn_r)

    mesh = plsc.VectorSubcoreMesh(core_axis_name="c", subcore_axis_name="s")
    partials = pl.kernel(
        functools.partial(_edge_body, n, npad, e, e_per_w),
        out_type=jax.ShapeDtypeStruct((NW, 3 * npad), jnp.float32),
        mesh=mesh,
        scratch_types=[
            pltpu.VMEM((4 * n,), jnp.float32),
            pltpu.VMEM((3 * npad,), jnp.float32),
            pltpu.VMEM((e_per_w,), jnp.int32),
            pltpu.VMEM((e_per_w,), jnp.int32),
            pltpu.SemaphoreType.DMA,
        ],
        compiler_params=pltpu.CompilerParams(needs_layout_passes=False),
    )(g.reshape(-1), ei_flat)

    out = pl.pallas_call(
        functools.partial(_final_body, npad, bn),
        grid=(grid,),
        in_specs=[
            pl.BlockSpec((bn, d), lambda i: (i, 0)),
            pl.BlockSpec((bn, 1), lambda i: (i, 0)),
            pl.BlockSpec(memory_space=pltpu.SMEM),
            pl.BlockSpec((NW, 3 * npad), lambda i: (0, 0)),
        ],
        out_specs=pl.BlockSpec((bn, d), lambda i: (i, 0)),
        out_shape=jax.ShapeDtypeStruct((n, d), jnp.float32),
    )(x, noise_x, bias, partials)
    return out
